# Initial kernel scaffold; baseline (speedup 1.0000x reference)
import jax
import jax.numpy as jnp
from jax.experimental import pallas as pl


def kernel(one_hot, id):
    raise NotImplementedError("write your pallas kernel here")


# speedup vs baseline: 1.7613x; 1.7613x over previous
import jax
import jax.numpy as jnp
from jax import lax
from jax.experimental import pallas as pl
from jax.experimental.pallas import tpu as pltpu
from jax.experimental.pallas import tpu_sc as plsc

_B, _N, _V = 8, 2048, 4096
_L = 16
_GCH = 16
_CW = _N // _GCH
_RPW = _N // _L


def _nonzero_mask_i32(vals):
    big = jnp.float32(1e38)
    return jnp.minimum(jnp.abs(vals) * big * big, jnp.float32(1.0)).astype(
        jnp.int32)


def _sc_body(rows_n_hbm, rows_t_hbm, nofp_hbm, table_hbm, col_out, idx_out,
             idxg_v, col_v, colT_v, nofp_v, destb_v, zero_v, cnts_v, offs_v,
             shbuf_v, sem):
    c = lax.axis_index("c")
    s = lax.axis_index("s")
    wid = s * 2 + c

    @pl.when(wid < _B)
    def _():
        b = wid
        lanes = lax.iota(jnp.int32, _L)

        pltpu.sync_copy(rows_n_hbm.at[b], idxg_v)
        handles = []
        for k in range(_GCH):
            handles.append(pltpu.async_copy(
                table_hbm.at[idxg_v.at[k]], col_v.at[pl.ds(k * _CW, _CW)],
                sem))
        for h in handles:
            h.wait()

        pltpu.sync_copy(rows_t_hbm.at[b], idxg_v)
        handles = []
        for k in range(_GCH):
            handles.append(pltpu.async_copy(
                table_hbm.at[idxg_v.at[k]], colT_v.at[pl.ds(k * _CW, _CW)],
                sem))
        for h in handles:
            h.wait()

        pltpu.sync_copy(nofp_hbm, nofp_v)

        def zbody(j, carry):
            zero_v[pl.ds(j * _L, _L)] = jnp.zeros((_L,), jnp.int32)
            return carry
        lax.fori_loop(0, _RPW, zbody, jnp.int32(0))
        cnts_v[...] = jnp.zeros((_L,), jnp.int32)

        def cntbody(j, carry):
            vals = colT_v[pl.ds(j * _L, _L)]
            cnts_v[...] = cnts_v[...] + _nonzero_mask_i32(vals)
            return carry
        lax.fori_loop(0, _RPW, cntbody, jnp.int32(0))

        cnts = cnts_v[...]
        shbuf_v[pl.ds(0, _L)] = jnp.zeros((_L,), jnp.int32)
        x = cnts
        for d in (1, 2, 4, 8):
            shbuf_v[pl.ds(_L, _L)] = x
            x = x + shbuf_v[pl.ds(_L - d, _L)]
        offs_v[...] = x - cnts + b * _N

        trash = _B * _N + lanes

        def dbody(j, carry):
            vals = colT_v[pl.ds(j * _L, _L)]
            m = _nonzero_mask_i32(vals)
            off = offs_v[...]
            dest = m * off + (1 - m) * trash
            destb_v[j // 8, pl.ds((j % 8) * _L, _L)] = dest
            offs_v[...] = off + m
            return carry
        lax.fori_loop(0, _RPW, dbody, jnp.int32(0))

        pltpu.sync_copy(zero_v, idx_out.at[pl.ds(b * _N, _N)])
        handles = []
        for k in range(_GCH):
            handles.append(pltpu.async_copy(
                nofp_v.at[k], idx_out.at[destb_v.at[k]], sem))
        for h in handles:
            h.wait()

        pltpu.sync_copy(col_v, col_out.at[pl.ds(b * _N, _N)])


@jax.jit
def kernel(one_hot, id):
    idc = jnp.asarray(id, jnp.int32)
    table = one_hot.reshape(_B * _N * _V)

    n_order = jnp.arange(_B * _N, dtype=jnp.int32)
    rows_n = (n_order * _V + idc).reshape(_B, _GCH, _CW)

    p = jnp.arange(_N, dtype=jnp.int32)
    nofp = (p % _L) * _RPW + p // _L
    bb = jnp.arange(_B, dtype=jnp.int32)[:, None]
    rows_t = ((bb * _N + nofp[None, :]) * _V + idc).reshape(_B, _GCH, _CW)
    nofp2d = nofp.reshape(_GCH, _CW)

    mesh = plsc.VectorSubcoreMesh(core_axis_name="c", subcore_axis_name="s")
    f = pl.kernel(
        _sc_body,
        mesh=mesh,
        out_type=[
            jax.ShapeDtypeStruct((_B * _N,), jnp.float32),
            jax.ShapeDtypeStruct((_B * _N + _L,), jnp.int32),
        ],
        scratch_types=[
            pltpu.VMEM((_GCH, _CW), jnp.int32),
            pltpu.VMEM((_N,), jnp.float32),
            pltpu.VMEM((_N,), jnp.float32),
            pltpu.VMEM((_GCH, _CW), jnp.int32),
            pltpu.VMEM((_GCH, _CW), jnp.int32),
            pltpu.VMEM((_N,), jnp.int32),
            pltpu.VMEM((_L,), jnp.int32),
            pltpu.VMEM((_L,), jnp.int32),
            pltpu.VMEM((2 * _L,), jnp.int32),
            pltpu.SemaphoreType.DMA,
        ],
    )
    col, idx = f(rows_n, rows_t, nofp2d, table)
    return (col.reshape(_B, _N, 1, 1), idx[:_B * _N].reshape(_B, _N))


# speedup vs baseline: 6.2428x; 3.5444x over previous
import jax
import jax.numpy as jnp
from jax import lax
from jax.experimental import pallas as pl
from jax.experimental.pallas import tpu as pltpu
from jax.experimental.pallas import tpu_sc as plsc

_B, _N, _V = 8, 2048, 4096
_L = 16
_GCH = 16
_CW = _N // _GCH
_RPW = _N // _L
_SEG = _N // _L


def _tc_body(ids_ref, x_ref, col3_ref, colf_ref):
    lane = ids_ref[0] % 128
    onehot = (lax.broadcasted_iota(jnp.int32, (1, 128), 1) == lane).astype(
        jnp.float32)
    colv = jnp.sum(x_ref[0] * onehot, axis=1)
    col3_ref[0, 0, :] = colv
    colf_ref[...] = colv


def _nonzero_mask_i32(vals):
    big = jnp.float32(1e38)
    return jnp.minimum(jnp.abs(vals) * big * big, jnp.float32(1.0)).astype(
        jnp.int32)


def _sc_body(col_hbm, idx_out, nvals_v, idxg_v, colT_v, destb_v, zero_v,
             cnts_v, offs_v, shbuf_v, sem):
    c = lax.axis_index("c")
    s = lax.axis_index("s")
    wid = s * 2 + c

    @pl.when(wid < _B)
    def _():
        b = wid
        lanes = lax.iota(jnp.int32, _L)
        lane128 = lanes * _SEG

        for k in range(_GCH):
            for u in range(8):
                nvals_v[k, pl.ds(u * _L, _L)] = lane128 + (k * 8 + u)
        base = b * _N
        for k in range(_GCH):
            for u in range(8):
                idxg_v[k, pl.ds(u * _L, _L)] = (
                    nvals_v[k, pl.ds(u * _L, _L)] + base)

        handles = []
        for k in range(_GCH):
            handles.append(pltpu.async_copy(
                col_hbm.at[idxg_v.at[k]], colT_v.at[pl.ds(k * _CW, _CW)],
                sem))
        for h in handles:
            h.wait()

        def zbody(j, carry):
            zero_v[pl.ds(j * _L, _L)] = jnp.zeros((_L,), jnp.int32)
            return carry
        lax.fori_loop(0, _RPW, zbody, jnp.int32(0))
        cnts_v[...] = jnp.zeros((_L,), jnp.int32)

        def cntbody(j, carry):
            vals = colT_v[pl.ds(j * _L, _L)]
            cnts_v[...] = cnts_v[...] + _nonzero_mask_i32(vals)
            return carry
        lax.fori_loop(0, _RPW, cntbody, jnp.int32(0))

        cnts = cnts_v[...]
        shbuf_v[pl.ds(0, _L)] = jnp.zeros((_L,), jnp.int32)
        x = cnts
        for d in (1, 2, 4, 8):
            shbuf_v[pl.ds(_L, _L)] = x
            x = x + shbuf_v[pl.ds(_L - d, _L)]
        offs_v[...] = x - cnts + base

        trash = _B * _N + lanes

        def dbody(j, carry):
            vals = colT_v[pl.ds(j * _L, _L)]
            m = _nonzero_mask_i32(vals)
            off = offs_v[...]
            dest = m * off + (1 - m) * trash
            destb_v[j // 8, pl.ds((j % 8) * _L, _L)] = dest
            offs_v[...] = off + m
            return carry
        lax.fori_loop(0, _RPW, dbody, jnp.int32(0))

        pltpu.sync_copy(zero_v, idx_out.at[pl.ds(base, _N)])
        handles = []
        for k in range(_GCH):
            handles.append(pltpu.async_copy(
                nvals_v.at[k], idx_out.at[destb_v.at[k]], sem))
        for h in handles:
            h.wait()


@jax.jit
def kernel(one_hot, id):
    ids = jnp.asarray(id, jnp.int32).reshape(1)

    grid_spec = pltpu.PrefetchScalarGridSpec(
        num_scalar_prefetch=1,
        grid=(_B,),
        in_specs=[
            pl.BlockSpec((1, _N, 128), lambda b, idr: (b, 0, idr[0] // 128)),
        ],
        out_specs=[
            pl.BlockSpec((1, 1, _N), lambda b, idr: (b, 0, 0)),
            pl.BlockSpec((_N,), lambda b, idr: (b,)),
        ],
    )
    col3, colf = pl.pallas_call(
        _tc_body,
        grid_spec=grid_spec,
        out_shape=[
            jax.ShapeDtypeStruct((_B, 1, _N), jnp.float32),
            jax.ShapeDtypeStruct((_B * _N,), jnp.float32),
        ],
    )(ids, one_hot)

    mesh = plsc.VectorSubcoreMesh(core_axis_name="c", subcore_axis_name="s")
    f = pl.kernel(
        _sc_body,
        mesh=mesh,
        out_type=jax.ShapeDtypeStruct((_B * _N + _L,), jnp.int32),
        scratch_types=[
            pltpu.VMEM((_GCH, _CW), jnp.int32),
            pltpu.VMEM((_GCH, _CW), jnp.int32),
            pltpu.VMEM((_N,), jnp.float32),
            pltpu.VMEM((_GCH, _CW), jnp.int32),
            pltpu.VMEM((_N,), jnp.int32),
            pltpu.VMEM((_L,), jnp.int32),
            pltpu.VMEM((_L,), jnp.int32),
            pltpu.VMEM((2 * _L,), jnp.int32),
            pltpu.SemaphoreType.DMA,
        ],
    )
    idx = f(colf)
    return (col3.reshape(_B, _N, 1, 1), idx[:_B * _N].reshape(_B, _N))
